# baseline (device time: 219390 ns/iter reference)
import jax
import jax.numpy as jnp
from jax import lax
from jax.experimental import pallas as pl
from jax.experimental.pallas import tpu as pltpu

N_DEV = 16
HEADS_PER = 8
SQ = 2048
DH = 128
DM = 1024
QB = 128
KB = 384
WINDOW = 128
SCALE = 0.08838834764831843
HALF = DM // 2
PC = SQ // 4
ZC = PC // 4

f32 = jnp.float32
bf16 = jnp.bfloat16


def _attn_body(x_ref, wq_ref, k_ref, v_ref, wo_ref, out_ref):
    qb = pl.program_id(0)
    h = pl.program_id(1)

    q0 = qb * QB
    start = pl.multiple_of(jnp.clip(q0 - WINDOW, 0, SQ - KB), WINDOW)
    k_win = k_ref[h, pl.ds(start, KB), :]
    v_win = v_ref[h, pl.ds(start, KB), :]

    q = jnp.dot(x_ref[...], wq_ref[h], preferred_element_type=f32)
    s = lax.dot_general(
        q.astype(bf16), k_win, (((1,), (1,)), ((), ())),
        preferred_element_type=f32,
    ) * SCALE
    qi = q0 + lax.broadcasted_iota(jnp.int32, (QB, KB), 0)
    ki = start + lax.broadcasted_iota(jnp.int32, (QB, KB), 1)
    e = jnp.exp(jnp.where(jnp.abs(qi - ki) <= WINDOW, s, -1e9))
    p = e / jnp.sum(e, axis=1, keepdims=True)
    ctx = jnp.dot(p.astype(bf16), v_win, preferred_element_type=f32)
    delta = jnp.dot(ctx.astype(bf16), wo_ref[h], preferred_element_type=f32)

    @pl.when(h == 0)
    def _():
        out_ref[...] = delta

    @pl.when(h != 0)
    def _():
        out_ref[...] += delta


def _attn(x2, wq_h, k_h, v_h, wo_h):
    return pl.pallas_call(
        _attn_body,
        grid=(SQ // QB, HEADS_PER),
        in_specs=[
            pl.BlockSpec((QB, DM), lambda qb, h: (qb, 0)),
            pl.BlockSpec((HEADS_PER, DM, DH), lambda qb, h: (0, 0, 0)),
            pl.BlockSpec((HEADS_PER, SQ, DH), lambda qb, h: (0, 0, 0)),
            pl.BlockSpec((HEADS_PER, SQ, DH), lambda qb, h: (0, 0, 0)),
            pl.BlockSpec((HEADS_PER, DH, DM), lambda qb, h: (0, 0, 0)),
        ],
        out_specs=pl.BlockSpec((QB, DM), lambda qb, h: (qb, 0)),
        out_shape=jax.ShapeDtypeStruct((SQ, DM), f32),
        compiler_params=pltpu.CompilerParams(
            dimension_semantics=("arbitrary", "arbitrary"),
        ),
    )(x2, wq_h, k_h, v_h, wo_h)


def _ar_body(in_ref, out_ref,
             sb_r, sb_l, rv1_r, rv1_l, mq_r, mq_l,
             zsb_r, zsb_l, zrv_r, zrv_l, zag_r, zag_l, pag_r, pag_l,
             p1s_r, p1r_r, p1s_l, p1r_l,
             zrs_s_r, zrs_r_r, zrs_s_l, zrs_r_l,
             zag_s_r, zag_r_r, zag_s_l, zag_r_l,
             p3s_r, p3r_r, p3s_l, p3r_l):
    i = lax.axis_index("i")
    p = lax.div(i, 4)
    j = lax.rem(i, 4)
    p4 = p * 4
    pright = p4 + lax.rem(j + 1, 4)
    pleft = p4 + lax.rem(j + 3, 4)
    zup = lax.rem(p + 1, 4) * 4 + j
    zdown = lax.rem(p + 3, 4) * 4 + j

    barrier_sem = pltpu.get_barrier_semaphore()
    for nbr in (pright, pleft, zup, zdown):
        pl.semaphore_signal(
            barrier_sem, inc=1,
            device_id=(nbr,), device_id_type=pl.DeviceIdType.MESH,
        )
    pl.semaphore_wait(barrier_sem, 4)

    def _rdma(src, dst, ssem, rsem, dev):
        return pltpu.make_async_remote_copy(
            src_ref=src, dst_ref=dst, send_sem=ssem, recv_sem=rsem,
            device_id=(dev,), device_id_type=pl.DeviceIdType.MESH,
        )

    for s in range(3):
        cr = lax.rem(j + 4 - s, 4)
        cl = lax.rem(j + s, 4)
        if s == 0:
            sb_r[...] = in_ref[cr][:, :HALF].astype(bf16)
            sb_l[...] = in_ref[cl][:, HALF:].astype(bf16)
        else:
            sb_r[...] = (in_ref[cr][:, :HALF]
                         + rv1_r[s - 1].astype(f32)).astype(bf16)
            sb_l[...] = (in_ref[cl][:, HALF:]
                         + rv1_l[s - 1].astype(f32)).astype(bf16)
        r = _rdma(sb_r, rv1_r.at[s], p1s_r.at[s], p1r_r.at[s], pright)
        l = _rdma(sb_l, rv1_l.at[s], p1s_l.at[s], p1r_l.at[s], pleft)
        r.start()
        l.start()
        r.wait()
        l.wait()

    qj_r = lax.rem(j + 1, 4)
    qj_l = lax.rem(j + 3, 4)
    mq_r[...] = in_ref[qj_r][:, :HALF] + rv1_r[2].astype(f32)
    mq_l[...] = in_ref[qj_l][:, HALF:] + rv1_l[2].astype(f32)

    for s in range(3):
        zr = lax.rem(p + 4 - s, 4)
        zl = lax.rem(p + s, 4)
        if s == 0:
            zsb_r[...] = mq_r[pl.ds(zr * ZC, ZC), :].astype(bf16)
            zsb_l[...] = mq_l[pl.ds(zl * ZC, ZC), :].astype(bf16)
        else:
            zsb_r[...] = (mq_r[pl.ds(zr * ZC, ZC), :]
                          + zrv_r[s - 1].astype(f32)).astype(bf16)
            zsb_l[...] = (mq_l[pl.ds(zl * ZC, ZC), :]
                          + zrv_l[s - 1].astype(f32)).astype(bf16)
        r = _rdma(zsb_r, zrv_r.at[s], zrs_s_r.at[s], zrs_r_r.at[s], zup)
        l = _rdma(zsb_l, zrv_l.at[s], zrs_s_l.at[s], zrs_r_l.at[s], zdown)
        r.start()
        l.start()
        r.wait()
        l.wait()

    zp_r = lax.rem(p + 1, 4)
    zp_l = lax.rem(p + 3, 4)
    zag_r[zp_r] = (mq_r[pl.ds(zp_r * ZC, ZC), :]
                   + zrv_r[2].astype(f32)).astype(bf16)
    zag_l[zp_l] = (mq_l[pl.ds(zp_l * ZC, ZC), :]
                   + zrv_l[2].astype(f32)).astype(bf16)

    for s in range(3):
        slot_r = lax.rem(p + 1 + 4 - s, 4)
        slot_l = lax.rem(p + 3 + s, 4)
        r = _rdma(zag_r.at[slot_r], zag_r.at[slot_r],
                  zag_s_r.at[s], zag_r_r.at[s], zup)
        l = _rdma(zag_l.at[slot_l], zag_l.at[slot_l],
                  zag_s_l.at[s], zag_r_l.at[s], zdown)
        r.start()
        l.start()
        r.wait()
        l.wait()

    pag_r[qj_r] = zag_r[...].reshape(PC, HALF)
    pag_l[qj_l] = zag_l[...].reshape(PC, HALF)

    for s in range(3):
        slot_r = lax.rem(j + 1 + 4 - s, 4)
        slot_l = lax.rem(j + 3 + s, 4)
        r = _rdma(pag_r.at[slot_r], pag_r.at[slot_r],
                  p3s_r.at[s], p3r_r.at[s], pright)
        l = _rdma(pag_l.at[slot_l], pag_l.at[slot_l],
                  p3s_l.at[s], p3r_l.at[s], pleft)
        r.start()
        l.start()
        r.wait()
        l.wait()

    out_ref[:, :, :HALF] = pag_r[...].astype(f32)
    out_ref[:, :, HALF:] = pag_l[...].astype(f32)


def _allreduce(partial):
    chunks = partial.reshape(4, PC, DM)
    sems = [pltpu.SemaphoreType.DMA((3,)) for _ in range(16)]
    out = pl.pallas_call(
        _ar_body,
        in_specs=[pl.BlockSpec(memory_space=pltpu.VMEM)],
        out_specs=pl.BlockSpec(memory_space=pltpu.VMEM),
        out_shape=jax.ShapeDtypeStruct((4, PC, DM), f32),
        scratch_shapes=[
            pltpu.VMEM((PC, HALF), bf16),
            pltpu.VMEM((PC, HALF), bf16),
            pltpu.VMEM((3, PC, HALF), bf16),
            pltpu.VMEM((3, PC, HALF), bf16),
            pltpu.VMEM((PC, HALF), f32),
            pltpu.VMEM((PC, HALF), f32),
            pltpu.VMEM((ZC, HALF), bf16),
            pltpu.VMEM((ZC, HALF), bf16),
            pltpu.VMEM((3, ZC, HALF), bf16),
            pltpu.VMEM((3, ZC, HALF), bf16),
            pltpu.VMEM((4, ZC, HALF), bf16),
            pltpu.VMEM((4, ZC, HALF), bf16),
            pltpu.VMEM((4, PC, HALF), bf16),
            pltpu.VMEM((4, PC, HALF), bf16),
        ] + sems,
        compiler_params=pltpu.CompilerParams(collective_id=0),
    )(chunks)
    return out.reshape(SQ, DM)


def kernel(x, Wq, K_ext, V_ext, Wo):
    i = lax.axis_index("i")
    h0 = i * HEADS_PER

    x2 = x[0].astype(bf16)
    k_h = lax.dynamic_slice_in_dim(K_ext[0], h0, HEADS_PER, axis=1)
    v_h = lax.dynamic_slice_in_dim(V_ext[0], h0, HEADS_PER, axis=1)
    k_h = jnp.transpose(k_h, (1, 0, 2)).astype(bf16)
    v_h = jnp.transpose(v_h, (1, 0, 2)).astype(bf16)
    wq_h = jnp.transpose(Wq.reshape(DM, HEADS_PER, DH), (1, 0, 2)).astype(bf16)
    wo_h = Wo.reshape(HEADS_PER, DH, DM).astype(bf16)

    partial = _attn(x2, wq_h, k_h, v_h, wo_h)
    out = _allreduce(partial)
    return out[None]


# device time: 168019 ns/iter; 1.3057x vs baseline; 1.3057x over previous
import jax
import jax.numpy as jnp
from jax import lax
from jax.experimental import pallas as pl
from jax.experimental.pallas import tpu as pltpu

N_DEV = 16
HEADS_PER = 8
SQ = 2048
DH = 128
DM = 1024
QB = 256
KB = 512
WINDOW = 128
SCALE = 0.08838834764831843
HALF = DM // 2
PC = SQ // 4
ZC = PC // 4

f32 = jnp.float32
bf16 = jnp.bfloat16


def _attn_body(x_ref, wq_ref, k_ref, v_ref, wo_ref, out_ref):
    qb = pl.program_id(0)
    h = pl.program_id(1)

    q0 = qb * QB
    start = pl.multiple_of(jnp.clip(q0 - WINDOW, 0, SQ - KB), WINDOW)
    k_win = k_ref[h, pl.ds(start, KB), :]
    v_win = v_ref[h, pl.ds(start, KB), :]

    q = jnp.dot(x_ref[...], wq_ref[h], preferred_element_type=f32)
    s = lax.dot_general(
        (q * SCALE).astype(bf16), k_win, (((1,), (1,)), ((), ())),
        preferred_element_type=f32,
    )
    qi = q0 + lax.broadcasted_iota(jnp.int32, (QB, KB), 0)
    ki = start + lax.broadcasted_iota(jnp.int32, (QB, KB), 1)
    e = jnp.exp(jnp.where(jnp.abs(qi - ki) <= WINDOW, s, -1e9))
    ctx = jnp.dot(e.astype(bf16), v_win, preferred_element_type=f32)
    ctx = ctx / jnp.sum(e, axis=1, keepdims=True)
    delta = jnp.dot(ctx.astype(bf16), wo_ref[h], preferred_element_type=f32)

    @pl.when(h == 0)
    def _():
        out_ref[...] = delta

    @pl.when(h != 0)
    def _():
        out_ref[...] += delta


def _attn(x2, wq_h, k_h, v_h, wo_h):
    return pl.pallas_call(
        _attn_body,
        grid=(SQ // QB, HEADS_PER),
        in_specs=[
            pl.BlockSpec((QB, DM), lambda qb, h: (qb, 0)),
            pl.BlockSpec((HEADS_PER, DM, DH), lambda qb, h: (0, 0, 0)),
            pl.BlockSpec((HEADS_PER, SQ, DH), lambda qb, h: (0, 0, 0)),
            pl.BlockSpec((HEADS_PER, SQ, DH), lambda qb, h: (0, 0, 0)),
            pl.BlockSpec((HEADS_PER, DH, DM), lambda qb, h: (0, 0, 0)),
        ],
        out_specs=pl.BlockSpec((QB, DM), lambda qb, h: (qb, 0)),
        out_shape=jax.ShapeDtypeStruct((SQ, DM), f32),
        compiler_params=pltpu.CompilerParams(
            dimension_semantics=("arbitrary", "arbitrary"),
        ),
    )(x2, wq_h, k_h, v_h, wo_h)


def _ar_body(in_ref, out_ref,
             sb_r, sb_l, rv1_r, rv1_l, mq_r, mq_l,
             zsb_r, zsb_l, zrv_r, zrv_l, zag_r, zag_l, pag_r, pag_l,
             p1s_r, p1r_r, p1s_l, p1r_l,
             zrs_s_r, zrs_r_r, zrs_s_l, zrs_r_l,
             zag_s_r, zag_r_r, zag_s_l, zag_r_l,
             p3s_r, p3r_r, p3s_l, p3r_l):
    i = lax.axis_index("i")
    p = lax.div(i, 4)
    j = lax.rem(i, 4)
    p4 = p * 4
    pright = p4 + lax.rem(j + 1, 4)
    pleft = p4 + lax.rem(j + 3, 4)
    zup = lax.rem(p + 1, 4) * 4 + j
    zdown = lax.rem(p + 3, 4) * 4 + j

    barrier_sem = pltpu.get_barrier_semaphore()
    for nbr in (pright, pleft, zup, zdown):
        pl.semaphore_signal(
            barrier_sem, inc=1,
            device_id=(nbr,), device_id_type=pl.DeviceIdType.MESH,
        )
    pl.semaphore_wait(barrier_sem, 4)

    def _rdma(src, dst, ssem, rsem, dev):
        return pltpu.make_async_remote_copy(
            src_ref=src, dst_ref=dst, send_sem=ssem, recv_sem=rsem,
            device_id=(dev,), device_id_type=pl.DeviceIdType.MESH,
        )

    for s in range(3):
        cr = lax.rem(j + 4 - s, 4)
        cl = lax.rem(j + s, 4)
        if s == 0:
            sb_r[...] = in_ref[cr][:, :HALF].astype(bf16)
            sb_l[...] = in_ref[cl][:, HALF:].astype(bf16)
        else:
            sb_r[...] = (in_ref[cr][:, :HALF]
                         + rv1_r[s - 1].astype(f32)).astype(bf16)
            sb_l[...] = (in_ref[cl][:, HALF:]
                         + rv1_l[s - 1].astype(f32)).astype(bf16)
        r = _rdma(sb_r, rv1_r.at[s], p1s_r.at[s], p1r_r.at[s], pright)
        l = _rdma(sb_l, rv1_l.at[s], p1s_l.at[s], p1r_l.at[s], pleft)
        r.start()
        l.start()
        r.wait()
        l.wait()

    qj_r = lax.rem(j + 1, 4)
    qj_l = lax.rem(j + 3, 4)
    mq_r[...] = in_ref[qj_r][:, :HALF] + rv1_r[2].astype(f32)
    mq_l[...] = in_ref[qj_l][:, HALF:] + rv1_l[2].astype(f32)

    for s in range(3):
        zr = lax.rem(p + 4 - s, 4)
        zl = lax.rem(p + s, 4)
        if s == 0:
            zsb_r[...] = mq_r[pl.ds(zr * ZC, ZC), :].astype(bf16)
            zsb_l[...] = mq_l[pl.ds(zl * ZC, ZC), :].astype(bf16)
        else:
            zsb_r[...] = (mq_r[pl.ds(zr * ZC, ZC), :]
                          + zrv_r[s - 1].astype(f32)).astype(bf16)
            zsb_l[...] = (mq_l[pl.ds(zl * ZC, ZC), :]
                          + zrv_l[s - 1].astype(f32)).astype(bf16)
        r = _rdma(zsb_r, zrv_r.at[s], zrs_s_r.at[s], zrs_r_r.at[s], zup)
        l = _rdma(zsb_l, zrv_l.at[s], zrs_s_l.at[s], zrs_r_l.at[s], zdown)
        r.start()
        l.start()
        r.wait()
        l.wait()

    zp_r = lax.rem(p + 1, 4)
    zp_l = lax.rem(p + 3, 4)
    zag_r[zp_r] = (mq_r[pl.ds(zp_r * ZC, ZC), :]
                   + zrv_r[2].astype(f32)).astype(bf16)
    zag_l[zp_l] = (mq_l[pl.ds(zp_l * ZC, ZC), :]
                   + zrv_l[2].astype(f32)).astype(bf16)

    for s in range(3):
        slot_r = lax.rem(p + 1 + 4 - s, 4)
        slot_l = lax.rem(p + 3 + s, 4)
        r = _rdma(zag_r.at[slot_r], zag_r.at[slot_r],
                  zag_s_r.at[s], zag_r_r.at[s], zup)
        l = _rdma(zag_l.at[slot_l], zag_l.at[slot_l],
                  zag_s_l.at[s], zag_r_l.at[s], zdown)
        r.start()
        l.start()
        r.wait()
        l.wait()

    pag_r[qj_r] = zag_r[...].reshape(PC, HALF)
    pag_l[qj_l] = zag_l[...].reshape(PC, HALF)

    for s in range(3):
        slot_r = lax.rem(j + 1 + 4 - s, 4)
        slot_l = lax.rem(j + 3 + s, 4)
        r = _rdma(pag_r.at[slot_r], pag_r.at[slot_r],
                  p3s_r.at[s], p3r_r.at[s], pright)
        l = _rdma(pag_l.at[slot_l], pag_l.at[slot_l],
                  p3s_l.at[s], p3r_l.at[s], pleft)
        r.start()
        l.start()
        r.wait()
        l.wait()

    out_ref[:, :, :HALF] = pag_r[...].astype(f32)
    out_ref[:, :, HALF:] = pag_l[...].astype(f32)


def _allreduce(partial):
    chunks = partial.reshape(4, PC, DM)
    sems = [pltpu.SemaphoreType.DMA((3,)) for _ in range(16)]
    out = pl.pallas_call(
        _ar_body,
        in_specs=[pl.BlockSpec(memory_space=pltpu.VMEM)],
        out_specs=pl.BlockSpec(memory_space=pltpu.VMEM),
        out_shape=jax.ShapeDtypeStruct((4, PC, DM), f32),
        scratch_shapes=[
            pltpu.VMEM((PC, HALF), bf16),
            pltpu.VMEM((PC, HALF), bf16),
            pltpu.VMEM((3, PC, HALF), bf16),
            pltpu.VMEM((3, PC, HALF), bf16),
            pltpu.VMEM((PC, HALF), f32),
            pltpu.VMEM((PC, HALF), f32),
            pltpu.VMEM((ZC, HALF), bf16),
            pltpu.VMEM((ZC, HALF), bf16),
            pltpu.VMEM((3, ZC, HALF), bf16),
            pltpu.VMEM((3, ZC, HALF), bf16),
            pltpu.VMEM((4, ZC, HALF), bf16),
            pltpu.VMEM((4, ZC, HALF), bf16),
            pltpu.VMEM((4, PC, HALF), bf16),
            pltpu.VMEM((4, PC, HALF), bf16),
        ] + sems,
        compiler_params=pltpu.CompilerParams(collective_id=0),
    )(chunks)
    return out.reshape(SQ, DM)


def kernel(x, Wq, K_ext, V_ext, Wo):
    i = lax.axis_index("i")
    h0 = i * HEADS_PER

    x2 = x[0].astype(bf16)
    k_h = lax.dynamic_slice_in_dim(K_ext[0], h0, HEADS_PER, axis=1)
    v_h = lax.dynamic_slice_in_dim(V_ext[0], h0, HEADS_PER, axis=1)
    k_h = jnp.transpose(k_h, (1, 0, 2)).astype(bf16)
    v_h = jnp.transpose(v_h, (1, 0, 2)).astype(bf16)
    wq_h = jnp.transpose(Wq.reshape(DM, HEADS_PER, DH), (1, 0, 2)).astype(bf16)
    wo_h = Wo.reshape(HEADS_PER, DH, DM).astype(bf16)

    partial = _attn(x2, wq_h, k_h, v_h, wo_h)
    out = _allreduce(partial)
    return out[None]
